# baseline (speedup 1.0000x reference)
import jax
import jax.numpy as jnp
from jax import lax
from jax.experimental import pallas as pl
from jax.experimental.pallas import tpu as pltpu
from jax.experimental.pallas import tpu_sc as plsc

N = 10000
E = 160000
D = 128
DO = 127

NT = 32
CPT = 320
WPC = 313
BMW = 100160
SPAN = E // 16
ECH = 2000
NECH = SPAN // ECH
LBW = 6144
RCAP = 22528
PCAP = 1024
OCAP = 2048
FLUSH = 512
ROWCAP = 321536
CNTP = 384
ACCR = 336

_i32 = jnp.int32


def _iota16():
  return lax.iota(_i32, 16)


def _splat(x):
  return jnp.zeros((16,), _i32) + x


def _append_ring(ref, ocur, vals, mask):
  rank = plsc.cumsum(jnp.ones((16,), _i32), mask=mask)
  pos = (ocur + rank - 1) & (OCAP - 1)
  plsc.store_scatter(ref, [pos], vals, mask=mask)
  return ocur + plsc.all_reduce_population_count(mask)


def _k1_body(col_hbm, src_hbm, dst_hbm, route_hbm, rlen_hbm,
             pairs_hbm, lens_hbm, cnt_hbm,
             col_v, bm, lbuf, pendb, ostage, cnt_v, sbuf, dbuf,
             hist, curv, loffv, rcurv, rlenv, lenv, semf):
  core = lax.axis_index("c")
  sub = lax.axis_index("s")
  wid = core * 16 + sub
  lo = wid * CPT
  hi = jnp.minimum(lo + CPT, N)
  loc = core * (16 * CPT)
  iot = _iota16()
  lane0 = iot == 0
  zeros16 = jnp.zeros((16,), _i32)
  ones16 = jnp.ones((16,), _i32)

  pltpu.sync_copy(col_hbm, col_v)

  def _zero_bm(i, _):
    bm[pl.ds(i * 16, 16)] = zeros16
    return 0
  lax.fori_loop(0, BMW // 16, _zero_bm, 0)

  def _zero_cnt(i, _):
    cnt_v[pl.ds(i * 16, 16)] = zeros16
    return 0
  lax.fori_loop(0, CNTP // 16, _zero_cnt, 0)
  rcurv[...] = zeros16

  def _self(i, ocur):
    k = i * 16 + iot
    m = k < (hi - lo)
    kc = jnp.where(m, k, 0)
    c = lo + kc
    word = kc * WPC + (c >> 5)
    bit = jnp.left_shift(_i32(1), c & 31)
    plsc.store_scatter(bm, [word], bit, mask=m)
    vn = plsc.load_gather(col_v, [c], mask=m)
    keep = m & (vn == 0.0)
    plsc.addupdate_scatter(cnt_v, [kc], ones16, mask=keep)
    return _append_ring(ostage, ocur, (c << 14) | c, mask=keep)
  ocur = lax.fori_loop(0, (CPT + 15) // 16, _self, zeros16)

  ebase = sub * SPAN

  def _owner(c):
    t = (c - loc) >> 6
    m = (c - loc).astype(jnp.uint32) < jnp.uint32(16 * CPT)
    o = ((t * 52429) >> 18) & 15
    return o, m

  def _chunkA(ch, _):
    off = pl.multiple_of(ebase + ch * ECH, 8)
    pltpu.sync_copy(src_hbm.at[pl.ds(off, ECH)], sbuf)
    pltpu.sync_copy(dst_hbm.at[pl.ds(off, ECH)], dbuf)
    hist[...] = zeros16

    def p1(j, _):
      s = sbuf[pl.ds(j * 16, 16)]
      d = dbuf[pl.ds(j * 16, 16)]
      for c in (s, d):
        o, m = _owner(c)
        rc, last = plsc.scan_count(o, mask=m)
        plsc.addupdate_scatter(hist, [o], rc, mask=m & last)
      return 0
    lax.fori_loop(0, ECH // 16, p1, 0)

    h = hist[...]
    hp = ((h + 127) >> 7) << 7
    lof = plsc.cumsum(hp) - hp
    loffv[...] = lof
    curv[...] = lof

    def p2(j, _):
      s = sbuf[pl.ds(j * 16, 16)]
      d = dbuf[pl.ds(j * 16, 16)]
      for c, n in ((s, d), (d, s)):
        o, m = _owner(c)
        rc, last = plsc.scan_count(o, mask=m)
        base = plsc.load_gather(curv, [o], mask=m)
        plsc.store_scatter(lbuf, [base + rc - 1], (c << 14) | n, mask=m)
        plsc.addupdate_scatter(curv, [o], rc, mask=m & last)
      return 0
    lax.fori_loop(0, ECH // 16, p2, 0)

    def flo(o, nd):
      ov = _splat(o)
      cs = plsc.load_gather(curv, [ov])
      ls = plsc.load_gather(loffv, [ov])
      cnt_o = jnp.max(cs - ls)
      npad = ((cnt_o + 127) >> 7) << 7
      olo = loc + o * CPT
      sent = _splat((olo << 14) | olo)
      start0 = jnp.max(cs)
      end0 = jnp.max(ls) + npad
      def fb_cond(st):
        return st < end0
      def fb_body(st):
        idx = st + iot
        plsc.store_scatter(lbuf, [idx], sent, mask=idx < end0)
        return st + 16
      lax.while_loop(fb_cond, fb_body, start0)
      rbase = ((core * 256 + sub * 16 + o) * RCAP
               + jnp.max(plsc.load_gather(rcurv, [ov])))
      lbase = jnp.max(ls)
      def fl_cond(st):
        fl, _nd2 = st
        return fl < npad
      def fl_body(st):
        fl, nd2 = st
        pltpu.async_copy(
            lbuf.at[pl.ds(pl.multiple_of(lbase + fl, 8), 128)],
            route_hbm.at[pl.ds(pl.multiple_of(rbase + fl, 8), 128)], semf)
        return fl + 128, nd2 + 1
      _fl, nd = lax.while_loop(fl_cond, fl_body, (_i32(0), nd))
      plsc.addupdate_scatter(rcurv, [ov], _splat(npad), mask=lane0)
      return nd

    nd = lax.fori_loop(0, 16, flo, _i32(0))

    def dr_cond(k):
      return k > 0
    def dr_body(k):
      pltpu.make_async_copy(route_hbm.at[pl.ds(0, 128)],
                            lbuf.at[pl.ds(0, 128)], semf).wait()
      return k - 1
    lax.while_loop(dr_cond, dr_body, nd)
    return 0

  lax.fori_loop(0, NECH, _chunkA, 0)

  rlenv[pl.ds(0, 16)] = rcurv[...]
  pltpu.sync_copy(rlenv,
                  rlen_hbm.at[pl.ds(pl.multiple_of(wid * 128, 128), 128)])
  plsc.subcore_barrier()

  pltpu.sync_copy(
      rlen_hbm.at[pl.ds(pl.multiple_of(core * 2048, 128), 2048)], pendb)
  mylens = plsc.load_gather(pendb, [iot * 128 + sub])

  def _consume(pcur, ocur):
    def body(i, oc):
      base = i * 16
      lv = (base + iot) < pcur
      p = pendb[pl.ds(base + PCAP, 16)]
      c = jnp.where(lv, p >> 14, lo)
      n = jnp.where(lv, p & 16383, 0)
      k = c - lo
      word = k * WPC + (n >> 5)
      bit = jnp.left_shift(_i32(1), n & 31)
      rc, _ = plsc.scan_count(p, mask=lv)
      first = lv & (rc == 1)
      w = plsc.load_gather(bm, [word], mask=lv)
      fresh = first & ((w & bit) == 0)
      def wave_cond(a):
        return jnp.max(a) > 0
      def wave_body(a):
        ab = a > 0
        rcw, _ = plsc.scan_count(word, mask=ab)
        lead = ab & (rcw == 1)
        plsc.addupdate_scatter(bm, [word], bit, mask=lead)
        return jnp.where(lead, 0, a)
      lax.while_loop(wave_cond, wave_body, jnp.where(fresh, 1, 0))
      vn = plsc.load_gather(col_v, [n], mask=lv)
      keep = fresh & (vn == 0.0)
      rc2, last2 = plsc.scan_count(k, mask=keep)
      plsc.addupdate_scatter(cnt_v, [k], rc2, mask=keep & last2)
      return _append_ring(ostage, oc, p, mask=keep)
    return lax.fori_loop(0, (pcur + 15) // 16, body, ocur)

  def _regions(p, carry):
    ocur, flushed = carry
    plen = jnp.max(jnp.where(iot == p, mylens, 0))
    rb = (core * 256 + p * 16 + sub) * RCAP

    def rd_cond(st):
      off, _oc, _fl = st
      return off < plen
    def rd_body(st):
      off, oc, fl = st
      pltpu.sync_copy(
          route_hbm.at[pl.ds(pl.multiple_of(rb + off, 8), PCAP)],
          pendb.at[pl.ds(PCAP, PCAP)])
      oc = _consume(jnp.minimum(plen - off, PCAP), oc)
      def fl_cond(s2):
        o2, f2 = s2
        return (o2 - f2) >= FLUSH
      def fl_body(s2):
        o2, f2 = s2
        pltpu.sync_copy(
            ostage.at[pl.ds(pl.multiple_of(f2 & (OCAP - 1), 512), FLUSH)],
            pairs_hbm.at[pl.ds(pl.multiple_of(wid * ROWCAP + f2, 512),
                               FLUSH)])
        return o2, f2 + FLUSH
      _o2, fl = lax.while_loop(fl_cond, fl_body, (jnp.max(oc), fl))
      return off + PCAP, oc, fl
    _off, ocur, flushed = lax.while_loop(rd_cond, rd_body,
                                         (_i32(0), ocur, flushed))
    return ocur, flushed

  ocur, flushed = lax.fori_loop(0, 16, _regions, (ocur, _i32(0)))

  pltpu.sync_copy(
      ostage.at[pl.ds(pl.multiple_of(flushed & (OCAP - 1), 512), FLUSH)],
      pairs_hbm.at[pl.ds(pl.multiple_of(wid * ROWCAP + flushed, 512),
                         FLUSH)])

  for j in range(8):
    lenv[pl.ds(j * 16, 16)] = ocur
  pltpu.sync_copy(lenv,
                  lens_hbm.at[pl.ds(pl.multiple_of(wid * 128, 128), 128)])
  pltpu.sync_copy(cnt_v,
                  cnt_hbm.at[pl.ds(pl.multiple_of(wid * CNTP, 128), CNTP)])


def _k2_body(x_hbm, zeros_hbm, pairs_hbm, lens_hbm, cnt_hbm, out_hbm,
             stage, pendb, accall, outall, cntv, lenv, acc_sh):
  core = lax.axis_index("c")
  sub = lax.axis_index("s")
  wid = core * 16 + sub
  lo = wid * CPT
  hi = jnp.minimum(lo + CPT, N)
  cw = hi - lo
  iot = _iota16()
  arow = sub * ACCR

  pltpu.sync_copy(zeros_hbm,
                  acc_sh.at[pl.ds(pl.multiple_of(arow, 16), ACCR)])

  pltpu.sync_copy(lens_hbm.at[pl.ds(pl.multiple_of(wid * 128, 128), 128)],
                  lenv)
  ln = jnp.max(lenv[pl.ds(0, 16)])
  pltpu.sync_copy(cnt_hbm.at[pl.ds(pl.multiple_of(wid * CNTP, 128), CNTP)],
                  cntv)

  def ch_cond(ch):
    return ch * 2048 < ln
  def ch_body(ch):
    pltpu.sync_copy(
        pairs_hbm.at[pl.ds(pl.multiple_of(wid * ROWCAP + ch * 2048, 512),
                           2048)], pendb)
    def batch(b, _):
      g0 = ch * 2048 + b * 16
      gm = (g0 + iot) < ln
      p = pendb[pl.ds(b * 16, 16)]
      n = jnp.where(gm, p & 16383, 0)
      k = jnp.where(gm, (p >> 14) - lo, CPT)
      pltpu.sync_copy(x_hbm.at[n], stage)
      pltpu.sync_copy(stage, acc_sh.at[arow + k], add=True)
      return 0
    lax.fori_loop(0, 128, batch, 0)
    return ch + 1
  lax.while_loop(ch_cond, ch_body, _i32(0))

  pltpu.sync_copy(acc_sh.at[pl.ds(pl.multiple_of(arow, 16), CPT)], accall)

  @plsc.parallel_loop(0, CPT, unroll=2)
  def _row(r):
    cs = plsc.load_gather(cntv, [_splat(r)])
    cf = cs.astype(jnp.float32)
    for j in range(D // 16):
      v = accall[r, pl.ds(j * 16, 16)]
      col = j * 16 + iot
      plsc.store_scatter(outall, [_splat(r), col], v / cf,
                         mask=col < DO)

  for t in range(CPT // 80):
    row0 = jnp.minimum(t * 80, cw - 80)
    pltpu.sync_copy(outall.at[pl.ds(pl.multiple_of(row0, 8), 80)],
                    out_hbm.at[pl.ds(pl.multiple_of(lo + row0, 8), 80)])


def _mesh():
  return plsc.VectorSubcoreMesh(core_axis_name="c", subcore_axis_name="s")


_CP = pltpu.CompilerParams(use_tc_tiling_on_sc=False,
                           needs_layout_passes=False)


@jax.jit
def kernel(x, edge_index):
  col = x[:, D - 1]
  k1 = pl.kernel(
      _k1_body,
      out_type=(
          jax.ShapeDtypeStruct((512 * RCAP,), _i32),
          jax.ShapeDtypeStruct((NT * 128,), _i32),
          jax.ShapeDtypeStruct((NT * ROWCAP,), _i32),
          jax.ShapeDtypeStruct((NT * 128,), _i32),
          jax.ShapeDtypeStruct((NT * CNTP,), _i32),
      ),
      mesh=_mesh(),
      compiler_params=_CP,
      scratch_types=[
          pltpu.VMEM((N,), jnp.float32),
          pltpu.VMEM((BMW,), _i32),
          pltpu.VMEM((LBW,), _i32),
          pltpu.VMEM((2 * PCAP,), _i32),
          pltpu.VMEM((OCAP,), _i32),
          pltpu.VMEM((CNTP,), _i32),
          pltpu.VMEM((ECH,), _i32),
          pltpu.VMEM((ECH,), _i32),
          pltpu.VMEM((16,), _i32),
          pltpu.VMEM((16,), _i32),
          pltpu.VMEM((16,), _i32),
          pltpu.VMEM((16,), _i32),
          pltpu.VMEM((128,), _i32),
          pltpu.VMEM((128,), _i32),
          pltpu.SemaphoreType.DMA,
      ],
  )
  _route, _rlens, pairs, lens, cnt = k1(col, edge_index[0], edge_index[1])

  k2 = pl.kernel(
      _k2_body,
      out_type=jax.ShapeDtypeStruct((N, DO), jnp.float32),
      mesh=_mesh(),
      compiler_params=_CP,
      scratch_types=[
          pltpu.VMEM((16, D), jnp.float32),
          pltpu.VMEM((2048,), _i32),
          pltpu.VMEM((CPT, D), jnp.float32),
          pltpu.VMEM((CPT, DO), jnp.float32),
          pltpu.VMEM((CNTP,), _i32),
          pltpu.VMEM((128,), _i32),
          pltpu.VMEM_SHARED((16 * ACCR, D), jnp.float32),
      ],
  )
  zeros = jnp.zeros((ACCR, D), jnp.float32)
  return k2(x, zeros, pairs, lens, cnt)
